# baseline (speedup 1.0000x reference)
import jax
import jax.numpy as jnp
from jax.experimental import pallas as pl
from jax.experimental.pallas import tpu as pltpu

_CONF = 0.25
_IOU = 0.7
_MAXDET = 300
_NCLS = 80
_N = 20000
_LANES = 128
_ROWS = 160
_NPAD = _ROWS * _LANES
_NEG = -1e30
_B = 4
_KROWS = 3
_OROWS = 304


def _nms_kernel(pred_ref, out_ref, ms_ref, x1_ref, y1_ref, x2_ref, y2_ref,
                a_ref, c_ref, kx1_ref, ky1_ref, kx2_ref, ky2_ref, ka_ref):
    rowmax0 = []
    for b in range(_B):
        x = pred_ref[b, 0]
        y = pred_ref[b, 1]
        w = pred_ref[b, 2] * 0.5
        h = pred_ref[b, 3] * 0.5
        x1 = x - w
        y1 = y - h
        x2 = x + w
        y2 = y + h
        s = pred_ref[b, 4]
        c = jnp.zeros_like(s)
        for i in range(1, _NCLS):
            v = pred_ref[b, 4 + i]
            c = jnp.where(v > s, float(i), c)
            s = jnp.maximum(s, v)
        x1_ref[b] = x1
        y1_ref[b] = y1
        x2_ref[b] = x2
        y2_ref[b] = y2
        a_ref[b] = (x2 - x1) * (y2 - y1)
        c_ref[b] = c
        ms = jnp.where(s > _CONF, s, _NEG)
        ms_ref[b] = ms
        rowmax0.append(jnp.transpose(jnp.max(ms, axis=1, keepdims=True)))
        out_ref[b] = jnp.zeros((_OROWS, _LANES), jnp.float32)
        z = jnp.zeros((_KROWS, _LANES), jnp.float32)
        kx1_ref[b] = z
        ky1_ref[b] = z
        kx2_ref[b] = z
        ky2_ref[b] = z
        ka_ref[b] = z

    lane128 = jax.lax.broadcasted_iota(jnp.int32, (1, _LANES), 1)
    iota160 = jax.lax.broadcasted_iota(jnp.int32, (1, _ROWS), 1)
    kiota = (jax.lax.broadcasted_iota(jnp.int32, (_KROWS, _LANES), 0) * _LANES
             + jax.lax.broadcasted_iota(jnp.int32, (_KROWS, _LANES), 1))
    big = jnp.int32(2 ** 30)

    def cond(carry):
        dones = carry[1]
        return jnp.logical_not(dones[0] & dones[1] & dones[2] & dones[3])

    def body(carry):
        counts, dones, rowmaxes = carry
        ncounts, ndones, nrowmaxes = [], [], []
        for b in range(_B):
            cnt, done, rowmax = counts[b], dones[b], rowmaxes[b]
            act = jnp.logical_not(done)
            m = jnp.max(rowmax)
            has = m > (_NEG * 0.5)
            exam = jnp.logical_and(act, has)
            r = jnp.min(jnp.where(rowmax == m, iota160, big))
            msrow = ms_ref[b, pl.ds(r, 1), :]
            l = jnp.min(jnp.where(msrow == m, lane128, big), keepdims=True)
            lm = lane128 == l
            bx1 = jnp.sum(jnp.where(lm, x1_ref[b, pl.ds(r, 1), :], 0.0),
                          keepdims=True)
            by1 = jnp.sum(jnp.where(lm, y1_ref[b, pl.ds(r, 1), :], 0.0),
                          keepdims=True)
            bx2 = jnp.sum(jnp.where(lm, x2_ref[b, pl.ds(r, 1), :], 0.0),
                          keepdims=True)
            by2 = jnp.sum(jnp.where(lm, y2_ref[b, pl.ds(r, 1), :], 0.0),
                          keepdims=True)
            bc = jnp.sum(jnp.where(lm, c_ref[b, pl.ds(r, 1), :], 0.0),
                         keepdims=True)
            barea = (bx2 - bx1) * (by2 - by1)

            inter = (jnp.maximum(
                jnp.minimum(bx2, kx2_ref[b]) - jnp.maximum(bx1, kx1_ref[b]), 0.0)
                * jnp.maximum(
                jnp.minimum(by2, ky2_ref[b]) - jnp.maximum(by1, ky1_ref[b]), 0.0))
            iou = inter / (ka_ref[b] + barea - inter + 1e-07)
            hit = jnp.logical_and(iou > _IOU, kiota < cnt)
            supp = jnp.max(jnp.where(hit, 1.0, 0.0)) > 0.0
            app = jnp.logical_and(exam, jnp.logical_not(supp))

            new_row = jnp.where(jnp.logical_and(lm, exam), _NEG, msrow)
            ms_ref[b, pl.ds(r, 1), :] = new_row
            nrm = jnp.max(new_row)
            nrowmaxes.append(
                jnp.where(jnp.logical_and(iota160 == r, exam), nrm, rowmax))

            kidx = jnp.where(app, cnt, jnp.int32(_KROWS * _LANES - 1))
            kr = kidx // _LANES
            kl = kidx - kr * _LANES
            klm = lane128 == kl
            kx1_ref[b, pl.ds(kr, 1), :] = jnp.where(
                klm, bx1, kx1_ref[b, pl.ds(kr, 1), :])
            ky1_ref[b, pl.ds(kr, 1), :] = jnp.where(
                klm, by1, ky1_ref[b, pl.ds(kr, 1), :])
            kx2_ref[b, pl.ds(kr, 1), :] = jnp.where(
                klm, bx2, kx2_ref[b, pl.ds(kr, 1), :])
            ky2_ref[b, pl.ds(kr, 1), :] = jnp.where(
                klm, by2, ky2_ref[b, pl.ds(kr, 1), :])
            ka_ref[b, pl.ds(kr, 1), :] = jnp.where(
                klm, barea, ka_ref[b, pl.ds(kr, 1), :])

            orow = jnp.where(
                lane128 == 0, bx1,
                jnp.where(lane128 == 1, by1,
                          jnp.where(lane128 == 2, bx2,
                                    jnp.where(lane128 == 3, by2,
                                              jnp.where(lane128 == 4, m,
                                                        jnp.where(lane128 == 5,
                                                                  bc, 0.0))))))
            oidx = jnp.where(app, cnt, jnp.int32(_MAXDET))
            out_ref[b, pl.ds(oidx, 1), :] = orow

            ncnt = cnt + jnp.where(app, 1, 0).astype(jnp.int32)
            ncounts.append(ncnt)
            ndones.append(done | jnp.logical_and(act, jnp.logical_not(has))
                          | (ncnt >= _MAXDET))
        return tuple(ncounts), tuple(ndones), tuple(nrowmaxes)

    zero = jnp.int32(0)
    f = jnp.bool_(False)
    jax.lax.while_loop(
        cond, body,
        ((zero,) * _B, (f,) * _B, tuple(rowmax0)))


def kernel(predictions):
    b = predictions.shape[0]
    pred = jnp.pad(predictions, ((0, 0), (0, _NPAD - _N), (0, 0)))
    pred = pred.transpose(0, 2, 1).reshape(b, 4 + _NCLS, _ROWS, _LANES)
    out = pl.pallas_call(
        _nms_kernel,
        out_shape=jax.ShapeDtypeStruct((b, _OROWS, _LANES), jnp.float32),
        scratch_shapes=[pltpu.VMEM((_B, _ROWS, _LANES), jnp.float32)] * 7
        + [pltpu.VMEM((_B, _KROWS, _LANES), jnp.float32)] * 5,
    )(pred)
    return out[:, :_MAXDET, :6]


# speedup vs baseline: 1.4395x; 1.4395x over previous
import jax
import jax.numpy as jnp
from jax.experimental import pallas as pl
from jax.experimental.pallas import tpu as pltpu

_CONF = 0.25
_IOU = 0.7
_MAXDET = 300
_NCLS = 80
_N = 20000
_LANES = 128
_ROWS = 160
_NPAD = _ROWS * _LANES
_NEG = -1e30
_B = 4


def _nms_kernel(pred_ref, out_ref, x1_ref, y1_ref, x2_ref, y2_ref, c_ref,
                area_ref):
    ms_init = []
    for b in range(_B):
        x = pred_ref[b, 0]
        y = pred_ref[b, 1]
        w = pred_ref[b, 2] * 0.5
        h = pred_ref[b, 3] * 0.5
        x1 = x - w
        y1 = y - h
        x2 = x + w
        y2 = y + h
        s = pred_ref[b, 4]
        c = jnp.zeros_like(s)
        for i in range(1, _NCLS):
            v = pred_ref[b, 4 + i]
            c = jnp.where(v > s, float(i), c)
            s = jnp.maximum(s, v)
        x1_ref[b] = x1
        y1_ref[b] = y1
        x2_ref[b] = x2
        y2_ref[b] = y2
        c_ref[b] = c
        area_ref[b] = (x2 - x1) * (y2 - y1)
        ms_init.append(jnp.where(s > _CONF, s, _NEG))

    rr = jax.lax.broadcasted_iota(jnp.int32, (_ROWS, _LANES), 0)
    ll = jax.lax.broadcasted_iota(jnp.int32, (_ROWS, _LANES), 1)
    ii = rr * _LANES + ll
    lane1 = jax.lax.broadcasted_iota(jnp.int32, (1, _LANES), 1)
    axes = (0, 1)

    def body(i, carry):
        new = []
        for b in range(_B):
            ms = carry[b]
            m = jnp.max(ms, axis=axes, keepdims=True)
            has = m > (_NEG * 0.5)
            idx = jnp.min(jnp.where(ms == m, ii, jnp.int32(2 ** 30)),
                          axis=axes, keepdims=True)
            pick = ii == idx
            x1 = x1_ref[b]
            y1 = y1_ref[b]
            x2 = x2_ref[b]
            y2 = y2_ref[b]
            bx1 = jnp.max(jnp.where(pick, x1, _NEG), axis=axes, keepdims=True)
            by1 = jnp.max(jnp.where(pick, y1, _NEG), axis=axes, keepdims=True)
            bx2 = jnp.max(jnp.where(pick, x2, _NEG), axis=axes, keepdims=True)
            by2 = jnp.max(jnp.where(pick, y2, _NEG), axis=axes, keepdims=True)
            bc = jnp.max(jnp.where(pick, c_ref[b], _NEG), axis=axes,
                         keepdims=True)

            inter = (jnp.maximum(jnp.minimum(bx2, x2) - jnp.maximum(bx1, x1), 0.0)
                     * jnp.maximum(jnp.minimum(by2, y2) - jnp.maximum(by1, y1), 0.0))
            a1 = (bx2 - bx1) * (by2 - by1)
            iou = inter / (a1 + area_ref[b] - inter + 1e-07)
            kill = jnp.logical_and(jnp.logical_or(iou > _IOU, pick), has)
            new.append(jnp.where(kill, _NEG, ms))

            valid = jnp.where(has, 1.0, 0.0)
            row = jnp.where(
                lane1 == 0, bx1,
                jnp.where(lane1 == 1, by1,
                          jnp.where(lane1 == 2, bx2,
                                    jnp.where(lane1 == 3, by2,
                                              jnp.where(lane1 == 4, m,
                                                        jnp.where(lane1 == 5, bc,
                                                                  0.0))))))
            out_ref[b, pl.ds(i, 1), :] = row * valid
        return tuple(new)

    jax.lax.fori_loop(0, _MAXDET, body, tuple(ms_init))


def kernel(predictions):
    b = predictions.shape[0]
    pred = jnp.pad(predictions, ((0, 0), (0, _NPAD - _N), (0, 0)))
    pred = pred.transpose(0, 2, 1).reshape(b, 4 + _NCLS, _ROWS, _LANES)
    out = pl.pallas_call(
        _nms_kernel,
        out_shape=jax.ShapeDtypeStruct((b, _MAXDET, _LANES), jnp.float32),
        scratch_shapes=[pltpu.VMEM((_B, _ROWS, _LANES), jnp.float32)] * 6,
    )(pred)
    return out[:, :, :6]


# speedup vs baseline: 2.8180x; 1.9576x over previous
import jax
import jax.numpy as jnp
from jax.experimental import pallas as pl
from jax.experimental.pallas import tpu as pltpu

_CONF = 0.25
_IOU = 0.7
_MAXDET = 300
_NCLS = 80
_N = 20000
_LANES = 128
_ROWS = 160
_NPAD = _ROWS * _LANES
_NEG = -1e30
_B = 4


def _nms_kernel(pred_ref, out_ref, ms_ref, x1_ref, y1_ref, x2_ref, y2_ref,
                c_ref, area_ref):
    for b in range(_B):
        x = pred_ref[b, 0]
        y = pred_ref[b, 1]
        w = pred_ref[b, 2] * 0.5
        h = pred_ref[b, 3] * 0.5
        x1 = x - w
        y1 = y - h
        x2 = x + w
        y2 = y + h
        s = pred_ref[b, 4]
        c = jnp.zeros_like(s)
        for i in range(1, _NCLS):
            v = pred_ref[b, 4 + i]
            c = jnp.where(v > s, float(i), c)
            s = jnp.maximum(s, v)
        x1_ref[b] = x1
        y1_ref[b] = y1
        x2_ref[b] = x2
        y2_ref[b] = y2
        c_ref[b] = c
        area_ref[b] = (x2 - x1) * (y2 - y1)
        ms_ref[b] = jnp.where(s > _CONF, s, _NEG)

    rr = jax.lax.broadcasted_iota(jnp.int32, (_B, _ROWS, _LANES), 1)
    ll = jax.lax.broadcasted_iota(jnp.int32, (_B, _ROWS, _LANES), 2)
    ii = rr * _LANES + ll
    lane1 = jax.lax.broadcasted_iota(jnp.int32, (1, _LANES), 1)
    axes = (1, 2)
    x1 = x1_ref[...]
    y1 = y1_ref[...]
    x2 = x2_ref[...]
    y2 = y2_ref[...]
    cc = c_ref[...]
    area = area_ref[...]

    def body(i, ms):
        m = jnp.max(ms, axis=axes, keepdims=True)
        has = m > (_NEG * 0.5)
        idx = jnp.min(jnp.where(ms == m, ii, jnp.int32(2 ** 30)),
                      axis=axes, keepdims=True)
        pick = ii == idx
        bx1 = jnp.max(jnp.where(pick, x1, _NEG), axis=axes, keepdims=True)
        by1 = jnp.max(jnp.where(pick, y1, _NEG), axis=axes, keepdims=True)
        bx2 = jnp.max(jnp.where(pick, x2, _NEG), axis=axes, keepdims=True)
        by2 = jnp.max(jnp.where(pick, y2, _NEG), axis=axes, keepdims=True)
        bc = jnp.max(jnp.where(pick, cc, _NEG), axis=axes, keepdims=True)

        inter = (jnp.maximum(jnp.minimum(bx2, x2) - jnp.maximum(bx1, x1), 0.0)
                 * jnp.maximum(jnp.minimum(by2, y2) - jnp.maximum(by1, y1), 0.0))
        a1 = (bx2 - bx1) * (by2 - by1)
        iou = inter / (a1 + area - inter + 1e-07)
        kill = jnp.logical_and(jnp.logical_or(iou > _IOU, pick), has)
        ms = jnp.where(kill, _NEG, ms)

        valid = jnp.where(has, 1.0, 0.0)
        row = jnp.where(
            lane1 == 0, bx1,
            jnp.where(lane1 == 1, by1,
                      jnp.where(lane1 == 2, bx2,
                                jnp.where(lane1 == 3, by2,
                                          jnp.where(lane1 == 4, m,
                                                    jnp.where(lane1 == 5, bc,
                                                              0.0))))))
        out_ref[:, pl.ds(i, 1), :] = row * valid
        return ms

    jax.lax.fori_loop(0, _MAXDET, body, ms_ref[...])


def kernel(predictions):
    b = predictions.shape[0]
    pred = jnp.pad(predictions, ((0, 0), (0, _NPAD - _N), (0, 0)))
    pred = pred.transpose(0, 2, 1).reshape(b, 4 + _NCLS, _ROWS, _LANES)
    out = pl.pallas_call(
        _nms_kernel,
        out_shape=jax.ShapeDtypeStruct((b, _MAXDET, _LANES), jnp.float32),
        scratch_shapes=[pltpu.VMEM((_B, _ROWS, _LANES), jnp.float32)] * 7,
    )(pred)
    return out[:, :, :6]


# speedup vs baseline: 3.0644x; 1.0874x over previous
import jax
import jax.numpy as jnp
from jax.experimental import pallas as pl
from jax.experimental.pallas import tpu as pltpu

_CONF = 0.25
_IOU = 0.7
_MAXDET = 300
_NCLS = 80
_N = 20000
_LANES = 128
_ROWS = 160
_NPAD = _ROWS * _LANES
_NEG = -1e30
_B = 4


def _nms_kernel(pred_ref, out_ref, ms_ref, x1_ref, y1_ref, x2_ref, y2_ref,
                c_ref, area_ref):
    for b in range(_B):
        x = pred_ref[b, 0]
        y = pred_ref[b, 1]
        w = pred_ref[b, 2] * 0.5
        h = pred_ref[b, 3] * 0.5
        x1 = x - w
        y1 = y - h
        x2 = x + w
        y2 = y + h
        s = pred_ref[b, 4]
        c = jnp.zeros_like(s)
        for i in range(1, _NCLS):
            v = pred_ref[b, 4 + i]
            c = jnp.where(v > s, float(i), c)
            s = jnp.maximum(s, v)
        x1_ref[b] = x1
        y1_ref[b] = y1
        x2_ref[b] = x2
        y2_ref[b] = y2
        c_ref[b] = c
        area_ref[b] = (x2 - x1) * (y2 - y1)
        ms_ref[b] = jnp.where(s > _CONF, s, _NEG)

    rr = jax.lax.broadcasted_iota(jnp.int32, (_B, _ROWS, _LANES), 1)
    ll = jax.lax.broadcasted_iota(jnp.int32, (_B, _ROWS, _LANES), 2)
    ii = rr * _LANES + ll
    lane1 = jax.lax.broadcasted_iota(jnp.int32, (1, _LANES), 1)
    axes = (1, 2)
    x1 = x1_ref[...]
    y1 = y1_ref[...]
    x2 = x2_ref[...]
    y2 = y2_ref[...]
    cc = c_ref[...]
    area = area_ref[...]

    def round_(i, ms):
        m = jnp.max(ms, axis=axes, keepdims=True)
        has = m > (_NEG * 0.5)
        idx = jnp.min(jnp.where(ms == m, ii, jnp.int32(2 ** 30)),
                      axis=axes, keepdims=True)
        pick = ii == idx
        bx1 = jnp.max(jnp.where(pick, x1, _NEG), axis=axes, keepdims=True)
        by1 = jnp.max(jnp.where(pick, y1, _NEG), axis=axes, keepdims=True)
        bx2 = jnp.max(jnp.where(pick, x2, _NEG), axis=axes, keepdims=True)
        by2 = jnp.max(jnp.where(pick, y2, _NEG), axis=axes, keepdims=True)
        bc = jnp.max(jnp.where(pick, cc, _NEG), axis=axes, keepdims=True)

        inter = (jnp.maximum(jnp.minimum(bx2, x2) - jnp.maximum(bx1, x1), 0.0)
                 * jnp.maximum(jnp.minimum(by2, y2) - jnp.maximum(by1, y1), 0.0))
        a1 = (bx2 - bx1) * (by2 - by1)
        iou = inter / (a1 + area - inter + 1e-07)
        kill = jnp.logical_and(jnp.logical_or(iou > _IOU, pick), has)
        ms = jnp.where(kill, _NEG, ms)

        valid = jnp.where(has, 1.0, 0.0)
        row = jnp.where(
            lane1 == 0, bx1,
            jnp.where(lane1 == 1, by1,
                      jnp.where(lane1 == 2, bx2,
                                jnp.where(lane1 == 3, by2,
                                          jnp.where(lane1 == 4, m,
                                                    jnp.where(lane1 == 5, bc,
                                                              0.0))))))
        out_ref[:, pl.ds(i, 1), :] = row * valid
        return ms

    def body(j, ms):
        ms = round_(j * 2, ms)
        return round_(j * 2 + 1, ms)

    jax.lax.fori_loop(0, _MAXDET // 2, body, ms_ref[...])


def kernel(predictions):
    b = predictions.shape[0]
    pred = jnp.pad(predictions, ((0, 0), (0, _NPAD - _N), (0, 0)))
    pred = pred.transpose(0, 2, 1).reshape(b, 4 + _NCLS, _ROWS, _LANES)
    out = pl.pallas_call(
        _nms_kernel,
        out_shape=jax.ShapeDtypeStruct((b, _MAXDET, _LANES), jnp.float32),
        scratch_shapes=[pltpu.VMEM((_B, _ROWS, _LANES), jnp.float32)] * 7,
    )(pred)
    return out[:, :, :6]


# speedup vs baseline: 3.1680x; 1.0338x over previous
import jax
import jax.numpy as jnp
from jax.experimental import pallas as pl
from jax.experimental.pallas import tpu as pltpu

_CONF = 0.25
_IOU = 0.7
_MAXDET = 300
_NCLS = 80
_N = 20000
_LANES = 128
_ROWS = 160
_NPAD = _ROWS * _LANES
_NEG = -1e30
_B = 4


def _nms_kernel(pred_ref, out_ref, ms_ref, x1_ref, y1_ref, x2_ref, y2_ref,
                c_ref, area_ref):
    for b in range(_B):
        x = pred_ref[b, 0]
        y = pred_ref[b, 1]
        w = pred_ref[b, 2] * 0.5
        h = pred_ref[b, 3] * 0.5
        x1 = x - w
        y1 = y - h
        x2 = x + w
        y2 = y + h
        s = pred_ref[b, 4]
        c = jnp.zeros_like(s)
        for i in range(1, _NCLS):
            v = pred_ref[b, 4 + i]
            c = jnp.where(v > s, float(i), c)
            s = jnp.maximum(s, v)
        x1_ref[b] = x1
        y1_ref[b] = y1
        x2_ref[b] = x2
        y2_ref[b] = y2
        c_ref[b] = c
        area_ref[b] = (x2 - x1) * (y2 - y1)
        ms_ref[b] = jnp.where(s > _CONF, s, _NEG)

    rr = jax.lax.broadcasted_iota(jnp.int32, (_B, _ROWS, _LANES), 1)
    ll = jax.lax.broadcasted_iota(jnp.int32, (_B, _ROWS, _LANES), 2)
    ii = rr * _LANES + ll
    lane1 = jax.lax.broadcasted_iota(jnp.int32, (1, _LANES), 1)
    axes = (1, 2)
    x1 = x1_ref[...]
    y1 = y1_ref[...]
    x2 = x2_ref[...]
    y2 = y2_ref[...]
    cc = c_ref[...]
    area = area_ref[...]

    def round_(i, ms):
        m = jnp.max(ms, axis=axes, keepdims=True)
        has = m > (_NEG * 0.5)
        idx = jnp.min(jnp.where(ms == m, ii, jnp.int32(2 ** 30)),
                      axis=axes, keepdims=True)
        pick = ii == idx
        bx1 = jnp.max(jnp.where(pick, x1, _NEG), axis=axes, keepdims=True)
        by1 = jnp.max(jnp.where(pick, y1, _NEG), axis=axes, keepdims=True)
        bx2 = jnp.max(jnp.where(pick, x2, _NEG), axis=axes, keepdims=True)
        by2 = jnp.max(jnp.where(pick, y2, _NEG), axis=axes, keepdims=True)
        bc = jnp.max(jnp.where(pick, cc, _NEG), axis=axes, keepdims=True)

        inter = (jnp.maximum(jnp.minimum(bx2, x2) - jnp.maximum(bx1, x1), 0.0)
                 * jnp.maximum(jnp.minimum(by2, y2) - jnp.maximum(by1, y1), 0.0))
        a1 = (bx2 - bx1) * (by2 - by1)
        iou = inter / (a1 + area - inter + 1e-07)
        kill = jnp.logical_and(jnp.logical_or(iou > _IOU, pick), has)
        ms = jnp.where(kill, _NEG, ms)

        valid = jnp.where(has, 1.0, 0.0)
        row = jnp.where(
            lane1 == 0, bx1,
            jnp.where(lane1 == 1, by1,
                      jnp.where(lane1 == 2, bx2,
                                jnp.where(lane1 == 3, by2,
                                          jnp.where(lane1 == 4, m,
                                                    jnp.where(lane1 == 5, bc,
                                                              0.0))))))
        out_ref[:, pl.ds(i, 1), :] = row * valid
        return ms

    def body(j, ms):
        for k in range(4):
            ms = round_(j * 4 + k, ms)
        return ms

    jax.lax.fori_loop(0, _MAXDET // 4, body, ms_ref[...])


def kernel(predictions):
    b = predictions.shape[0]
    pred = jnp.pad(predictions, ((0, 0), (0, _NPAD - _N), (0, 0)))
    pred = pred.transpose(0, 2, 1).reshape(b, 4 + _NCLS, _ROWS, _LANES)
    out = pl.pallas_call(
        _nms_kernel,
        out_shape=jax.ShapeDtypeStruct((b, _MAXDET, _LANES), jnp.float32),
        scratch_shapes=[pltpu.VMEM((_B, _ROWS, _LANES), jnp.float32)] * 7,
    )(pred)
    return out[:, :, :6]
